# baseline (speedup 1.0000x reference)
import functools

import jax
import jax.numpy as jnp
from jax import lax
from jax.experimental import pallas as pl
from jax.experimental.pallas import tpu as pltpu
from jax.experimental.pallas import tpu_sc as plsc

D_MODEL = 1024
SCALE = float(D_MODEL) ** 0.5

_INFO = plsc.get_sparse_core_info()
NC, NS, L = _INFO.num_cores, _INFO.num_subcores, _INFO.num_lanes
NW = NC * NS

N_TOK = 4 * 8192
RPW = N_TOK // NW
C = 16
NCH = RPW // C
NBUF = 4


def _body(ids_hbm, table_hbm, out_hbm,
          idx_v, buf0, buf1, buf2, buf3,
          gsem0, gsem1, gsem2, gsem3, osem0, osem1, osem2, osem3):
    bufs = (buf0, buf1, buf2, buf3)
    gsems = (gsem0, gsem1, gsem2, gsem3)
    osems = (osem0, osem1, osem2, osem3)

    wid = lax.axis_index("s") * NC + lax.axis_index("c")
    base = pl.multiple_of(wid * RPW, RPW)
    pltpu.sync_copy(ids_hbm.at[pl.ds(base, RPW)], idx_v)

    def gather(g, b):
        off = pl.multiple_of(g * C, C)
        pltpu.async_copy(table_hbm.at[idx_v.at[pl.ds(off, C)]], bufs[b],
                         gsems[b])

    def wait_gather(b):
        pltpu.make_async_copy(out_hbm.at[pl.ds(0, C)], bufs[b],
                              gsems[b]).wait()

    def wait_out(b):
        pltpu.make_async_copy(out_hbm.at[pl.ds(0, C)], bufs[b],
                              osems[b]).wait()

    def scale(b):
        def row(r, carry):
            for j in range(D_MODEL // L):
                sl = pl.ds(j * L, L)
                bufs[b][r, sl] = bufs[b][r, sl] * SCALE
            return carry

        lax.fori_loop(0, C, row, 0)

    def writeback(g, b):
        off = pl.multiple_of(g * C, C)
        pltpu.async_copy(bufs[b], out_hbm.at[pl.ds(base + off, C)],
                         osems[b])

    gather(0, 0)
    gather(1, 1)

    def outer(go, carry):
        for j in range(NBUF):
            g = go * NBUF + j
            wait_gather(j)
            scale(j)
            writeback(g, j)
            nb = (j + 2) % NBUF
            @pl.when(g + 2 < NCH)
            def _():
                @pl.when(g >= 2)
                def _():
                    wait_out(nb)
                gather(g + 2, nb)
        return carry

    lax.fori_loop(0, NCH // NBUF, outer, 0)

    for b in range(NBUF):
        wait_out(b)


_encoder = functools.partial(
    pl.kernel,
    out_type=jax.ShapeDtypeStruct((N_TOK, D_MODEL), jnp.float32),
    mesh=plsc.VectorSubcoreMesh(core_axis_name="c", subcore_axis_name="s"),
    scratch_types=[
        pltpu.VMEM((RPW,), jnp.int32),
        pltpu.VMEM((C, D_MODEL), jnp.float32),
        pltpu.VMEM((C, D_MODEL), jnp.float32),
        pltpu.VMEM((C, D_MODEL), jnp.float32),
        pltpu.VMEM((C, D_MODEL), jnp.float32),
        pltpu.SemaphoreType.DMA,
        pltpu.SemaphoreType.DMA,
        pltpu.SemaphoreType.DMA,
        pltpu.SemaphoreType.DMA,
        pltpu.SemaphoreType.DMA,
        pltpu.SemaphoreType.DMA,
        pltpu.SemaphoreType.DMA,
        pltpu.SemaphoreType.DMA,
    ],
)(_body)


def kernel(input_ids, embedding_weight):
    ids = input_ids.reshape(-1).astype(jnp.int32)
    out = _encoder(ids, embedding_weight)
    return out.reshape(*input_ids.shape, D_MODEL)


# speedup vs baseline: 1.0672x; 1.0672x over previous
import functools

import jax
import jax.numpy as jnp
from jax import lax
from jax.experimental import pallas as pl
from jax.experimental.pallas import tpu as pltpu
from jax.experimental.pallas import tpu_sc as plsc

D_MODEL = 1024
SCALE = float(D_MODEL) ** 0.5

_INFO = plsc.get_sparse_core_info()
NC, NS, L = _INFO.num_cores, _INFO.num_subcores, _INFO.num_lanes
NW = NC * NS

N_TOK = 4 * 8192
RPW = N_TOK // NW
C = 16
NCH = RPW // C
NBUF = 4


def _body(ids_hbm, table_hbm, out_hbm,
          idx_v, buf0, buf1, buf2, buf3,
          gsem0, gsem1, gsem2, gsem3, osem0, osem1, osem2, osem3):
    bufs = (buf0, buf1, buf2, buf3)
    gsems = (gsem0, gsem1, gsem2, gsem3)
    osems = (osem0, osem1, osem2, osem3)

    wid = lax.axis_index("s") * NC + lax.axis_index("c")
    base = pl.multiple_of(wid * RPW, RPW)
    pltpu.sync_copy(ids_hbm.at[pl.ds(base, RPW)], idx_v)

    def gather(g, b):
        off = pl.multiple_of(g * C, C)
        pltpu.async_copy(table_hbm.at[idx_v.at[pl.ds(off, C)]], bufs[b],
                         gsems[b])

    def wait_gather(b):
        pltpu.make_async_copy(out_hbm.at[pl.ds(0, C)], bufs[b],
                              gsems[b]).wait()

    def wait_out(b):
        pltpu.make_async_copy(out_hbm.at[pl.ds(0, C)], bufs[b],
                              osems[b]).wait()

    def scale(b):
        def row(r, carry):
            for j in range(D_MODEL // L):
                sl = pl.ds(j * L, L)
                bufs[b][r, sl] = bufs[b][r, sl] * SCALE
            return carry

        lax.fori_loop(0, C, row, 0)

    def writeback(g, b):
        off = pl.multiple_of(g * C, C)
        pltpu.async_copy(bufs[b], out_hbm.at[pl.ds(base + off, C)],
                         osems[b])

    gather(0, 0)
    gather(1, 1)

    def outer(go, carry):
        for j in range(NBUF):
            g = go * NBUF + j
            wait_gather(j)
            nb = (j + 2) % NBUF
            @pl.when(g + 2 < NCH)
            def _():
                @pl.when(g >= 2)
                def _():
                    wait_out(nb)
                gather(g + 2, nb)
            scale(j)
            writeback(g, j)
        return carry

    lax.fori_loop(0, NCH // NBUF, outer, 0)

    for b in range(NBUF):
        wait_out(b)


_encoder = functools.partial(
    pl.kernel,
    out_type=jax.ShapeDtypeStruct((N_TOK, D_MODEL), jnp.float32),
    mesh=plsc.VectorSubcoreMesh(core_axis_name="c", subcore_axis_name="s"),
    scratch_types=[
        pltpu.VMEM((RPW,), jnp.int32),
        pltpu.VMEM((C, D_MODEL), jnp.float32),
        pltpu.VMEM((C, D_MODEL), jnp.float32),
        pltpu.VMEM((C, D_MODEL), jnp.float32),
        pltpu.VMEM((C, D_MODEL), jnp.float32),
        pltpu.SemaphoreType.DMA,
        pltpu.SemaphoreType.DMA,
        pltpu.SemaphoreType.DMA,
        pltpu.SemaphoreType.DMA,
        pltpu.SemaphoreType.DMA,
        pltpu.SemaphoreType.DMA,
        pltpu.SemaphoreType.DMA,
        pltpu.SemaphoreType.DMA,
    ],
)(_body)


def kernel(input_ids, embedding_weight):
    ids = input_ids.reshape(-1).astype(jnp.int32)
    out = _encoder(ids, embedding_weight)
    return out.reshape(*input_ids.shape, D_MODEL)


# speedup vs baseline: 1.0695x; 1.0022x over previous
import functools

import jax
import jax.numpy as jnp
from jax import lax
from jax.experimental import pallas as pl
from jax.experimental.pallas import tpu as pltpu
from jax.experimental.pallas import tpu_sc as plsc

D_MODEL = 1024
SCALE = float(D_MODEL) ** 0.5

_INFO = plsc.get_sparse_core_info()
NC, NS, L = _INFO.num_cores, _INFO.num_subcores, _INFO.num_lanes
NW = NC * NS

N_TOK = 4 * 8192
RPW = N_TOK // NW
C = 16
NCH = RPW // C
NBUF = 4


def _body(ids_hbm, table_hbm, out_hbm,
          idx_v, buf0, buf1, buf2, buf3,
          gsem0, gsem1, gsem2, gsem3, osem0, osem1, osem2, osem3):
    bufs = (buf0, buf1, buf2, buf3)
    gsems = (gsem0, gsem1, gsem2, gsem3)
    osems = (osem0, osem1, osem2, osem3)

    wid = lax.axis_index("s") * NC + lax.axis_index("c")
    base = pl.multiple_of(wid * RPW, RPW)
    pltpu.sync_copy(ids_hbm.at[pl.ds(base, RPW)], idx_v)

    def gather(g, b):
        off = pl.multiple_of(g * C, C)
        pltpu.async_copy(table_hbm.at[idx_v.at[pl.ds(off, C)]], bufs[b],
                         gsems[b])

    def wait_gather(b):
        pltpu.make_async_copy(out_hbm.at[pl.ds(0, C)], bufs[b],
                              gsems[b]).wait()

    def wait_out(b):
        pltpu.make_async_copy(out_hbm.at[pl.ds(0, C)], bufs[b],
                              osems[b]).wait()

    def scale(b):
        @plsc.parallel_loop(0, C, step=1)
        def row(r):
            for j in range(D_MODEL // L):
                sl = pl.ds(j * L, L)
                bufs[b][r, sl] = bufs[b][r, sl] * SCALE

    def writeback(g, b):
        off = pl.multiple_of(g * C, C)
        pltpu.async_copy(bufs[b], out_hbm.at[pl.ds(base + off, C)],
                         osems[b])

    gather(0, 0)
    gather(1, 1)

    def outer(go, carry):
        for j in range(NBUF):
            g = go * NBUF + j
            wait_gather(j)
            nb = (j + 2) % NBUF
            @pl.when(g + 2 < NCH)
            def _():
                @pl.when(g >= 2)
                def _():
                    wait_out(nb)
                gather(g + 2, nb)
            scale(j)
            writeback(g, j)
        return carry

    lax.fori_loop(0, NCH // NBUF, outer, 0)

    for b in range(NBUF):
        wait_out(b)


_encoder = functools.partial(
    pl.kernel,
    out_type=jax.ShapeDtypeStruct((N_TOK, D_MODEL), jnp.float32),
    mesh=plsc.VectorSubcoreMesh(core_axis_name="c", subcore_axis_name="s"),
    scratch_types=[
        pltpu.VMEM((RPW,), jnp.int32),
        pltpu.VMEM((C, D_MODEL), jnp.float32),
        pltpu.VMEM((C, D_MODEL), jnp.float32),
        pltpu.VMEM((C, D_MODEL), jnp.float32),
        pltpu.VMEM((C, D_MODEL), jnp.float32),
        pltpu.SemaphoreType.DMA,
        pltpu.SemaphoreType.DMA,
        pltpu.SemaphoreType.DMA,
        pltpu.SemaphoreType.DMA,
        pltpu.SemaphoreType.DMA,
        pltpu.SemaphoreType.DMA,
        pltpu.SemaphoreType.DMA,
        pltpu.SemaphoreType.DMA,
    ],
)(_body)


def kernel(input_ids, embedding_weight):
    ids = input_ids.reshape(-1).astype(jnp.int32)
    out = _encoder(ids, embedding_weight)
    return out.reshape(*input_ids.shape, D_MODEL)
